# Initial kernel scaffold; baseline (speedup 1.0000x reference)
import jax
import jax.numpy as jnp
from jax.experimental import pallas as pl


def kernel(feature_index, feature_value, feature_weight, interaction_weight, bias):
    raise NotImplementedError("write your pallas kernel here")


# speedup vs baseline: 1.6708x; 1.6708x over previous
import functools

import jax
import jax.numpy as jnp
from jax import lax
from jax.experimental import pallas as pl
from jax.experimental.pallas import tpu as pltpu
from jax.experimental.pallas import tpu_sc as plsc

B, F, V, D = 16384, 26, 1000000, 32
NC, NS = 2, 16
NW = NC * NS
RPW = B // NW
WB = 4
NWIN = RPW // WB
WIDX = WB * F
WPAD = 112
L = 16
NCHUNK = WPAD // L

_mesh = plsc.VectorSubcoreMesh(core_axis_name="c", subcore_axis_name="s")

_cp = pltpu.CompilerParams(needs_layout_passes=False,
                           use_tc_tiling_on_sc=False)


@functools.partial(
    pl.kernel,
    compiler_params=_cp,
    out_type=jax.ShapeDtypeStruct((NW, NWIN, L), jnp.float32),
    mesh=_mesh,
    scratch_types=[
        pltpu.VMEM((NWIN, WIDX), jnp.int32),
        pltpu.VMEM((NWIN, WPAD), jnp.float32),
        pltpu.VMEM((WIDX, D), jnp.float32),
        pltpu.VMEM((WPAD,), jnp.float32),
        pltpu.VMEM((L,), jnp.float32),
        pltpu.VMEM((NWIN, L), jnp.float32),
    ],
)
def _fm_sc(fi_hbm, fv_hbm, iw_hbm, fw_hbm, bias_hbm, out_hbm,
           idx_v, fv_v, row_v, fw_v, bias_v, out_v):
    wid = lax.axis_index("s") * NC + lax.axis_index("c")
    pltpu.sync_copy(fi_hbm.at[wid], idx_v)
    pltpu.sync_copy(fv_hbm.at[wid], fv_v)
    pltpu.sync_copy(bias_hbm, bias_v.at[pl.ds(0, 1)])
    bias_val = bias_v[pl.ds(0, L)][0]
    lane = lax.iota(jnp.int32, L)
    zero_vec = jnp.zeros((L,), jnp.float32)

    @pl.loop(0, NWIN)
    def _window(w):
        pltpu.sync_copy(iw_hbm.at[idx_v.at[w]], row_v)
        pltpu.sync_copy(fw_hbm.at[idx_v.at[w]], fw_v.at[pl.ds(0, WIDX)])
        fvc = [fv_v[w, pl.ds(c * L, L)] for c in range(NCHUNK)]
        fwc = [fw_v[pl.ds(c * L, L)] for c in range(NCHUNK)]
        res_vec = zero_vec
        for r in range(WB):
            acc0 = jnp.zeros((L,), jnp.float32)
            acc1 = jnp.zeros((L,), jnp.float32)
            sq0 = jnp.zeros((L,), jnp.float32)
            sq1 = jnp.zeros((L,), jnp.float32)
            fo = jnp.float32(0.0)
            for f in range(F):
                j = r * F + f
                v = fvc[j // L][j % L]
                fo = fo + v * fwc[j // L][j % L]
                e0 = v * row_v[j, pl.ds(0, L)]
                e1 = v * row_v[j, pl.ds(L, L)]
                acc0 = acc0 + e0
                acc1 = acc1 + e1
                sq0 = sq0 + e0 * e0
                sq1 = sq1 + e1 * e1
            dvec = (acc0 * acc0 - sq0) + (acc1 * acc1 - sq1)
            res = 0.5 * jnp.sum(dvec) + fo + bias_val
            res_vec = jnp.where(lane == r, res, res_vec)
        out_v[w] = res_vec

    pltpu.sync_copy(out_v, out_hbm.at[wid])


def kernel(feature_index, feature_value, feature_weight, interaction_weight,
           bias):
    fi = feature_index.astype(jnp.int32).reshape(NW, NWIN, WIDX)
    fv = feature_value.reshape(NW, NWIN, WIDX)
    fv = jnp.pad(fv, ((0, 0), (0, 0), (0, WPAD - WIDX)))
    fw = feature_weight.reshape(V)
    out = _fm_sc(fi, fv, interaction_weight, fw, bias)
    return out[:, :, :WB].reshape(B, 1)
